# Initial kernel scaffold; baseline (speedup 1.0000x reference)
import jax
import jax.numpy as jnp
from jax.experimental import pallas as pl


def kernel(x, edges_uv, W1, b1, W2, b2, W3, b3):
    raise NotImplementedError("write your pallas kernel here")


# speedup vs baseline: 1.9070x; 1.9070x over previous
import functools

import jax
import jax.numpy as jnp
import numpy as np
from jax import lax
from jax.experimental import pallas as pl
from jax.experimental.pallas import tpu as pltpu
from jax.experimental.pallas import tpu_sc as plsc

_NC, _NS, _LANES = 2, 16, 16
_NW = _NC * _NS
_CHUNK = 512
_IDXG = 128


def _proj_body(x_ref, w1_ref, b1_ref, pu_ref, pv_ref):
    d = x_ref.shape[1]
    w1 = w1_ref[...]
    wu = w1[0:d] + w1[2 * d:3 * d]
    wv = w1[d:2 * d] - w1[2 * d:3 * d]
    xb = x_ref[...]
    pu_ref[...] = (
        jnp.dot(xb, wu, preferred_element_type=jnp.float32) + b1_ref[...]
    )
    pv_ref[...] = jnp.dot(xb, wv, preferred_element_type=jnp.float32)


def _sc_gather_body(n_chunks_tot, u_hbm, v_hbm, pu_hbm, pv_hbm,
                    gu_hbm, gv_hbm, ng_hbm,
                    u_v, v_v, iu_v, iv_v, ng_v, gu_v, gv_v, sem):
    wid = lax.axis_index("s") * _NC + lax.axis_index("c")
    n_floor = n_chunks_tot // _NW
    n_rem = n_chunks_tot % _NW
    n_w = n_floor + jnp.where(wid < n_rem, 1, 0)

    def chunk_body(k, _):
        off = (wid + k * _NW) * _CHUNK
        pltpu.sync_copy(u_hbm.at[pl.ds(off, _CHUNK)], u_v)
        pltpu.sync_copy(v_hbm.at[pl.ds(off, _CHUNK)], v_v)

        def lane_body(i, _):
            s = pl.ds(i * _LANES, _LANES)
            uu = u_v[s]
            vv = v_v[s]
            iu_v[s] = jnp.minimum(uu, vv)
            iv_v[s] = jnp.maximum(uu, vv)
            ng_v[s] = jnp.where(uu > vv, -1.0, 1.0)
            return 0

        lax.fori_loop(0, _CHUNK // _LANES, lane_body, 0)

        cps = []
        for j in range(_CHUNK // _IDXG):
            s = pl.ds(j * _IDXG, _IDXG)
            cps.append(pltpu.async_copy(pu_hbm.at[iu_v.at[s]], gu_v.at[s], sem))
            cps.append(pltpu.async_copy(pv_hbm.at[iv_v.at[s]], gv_v.at[s], sem))
        for cp in cps:
            cp.wait()

        pltpu.sync_copy(gu_v, gu_hbm.at[pl.ds(off, _CHUNK)])
        pltpu.sync_copy(gv_v, gv_hbm.at[pl.ds(off, _CHUNK)])
        pltpu.sync_copy(ng_v, ng_hbm.at[pl.ds(off, _CHUNK)])
        return 0

    lax.fori_loop(0, n_w, chunk_body, 0)


def _mlp_body(gu_ref, gv_ref, ng_ref, w2_ref, b2_ref, w3_ref, b3_ref,
              pt_ref, out_ref):
    h1 = jnp.tanh(gu_ref[...] + gv_ref[...])
    h2 = jnp.tanh(
        jnp.dot(h1, w2_ref[...], preferred_element_type=jnp.float32)
        + b2_ref[...]
    )
    y = (jnp.dot(h2, w3_ref[...], preferred_element_type=jnp.float32)
         + b3_ref[...])
    s = (1.5 * ng_ref[...]) * jnp.tanh(y)
    out_ref[...] = s - jnp.dot(s, pt_ref[...],
                               preferred_element_type=jnp.float32)


def kernel(x, edges_uv, W1, b1, W2, b2, W3, b3):
    n, d = x.shape
    e = edges_uv.shape[0]
    h = W2.shape[0]
    kk = W3.shape[1]
    k = int(np.sqrt(kk))

    u = edges_uv[:, 0]
    v = edges_uv[:, 1]

    pu, pv = pl.pallas_call(
        _proj_body,
        out_shape=(
            jax.ShapeDtypeStruct((n, h), jnp.float32),
            jax.ShapeDtypeStruct((n, h), jnp.float32),
        ),
    )(x, W1, b1.reshape(1, h))

    n_chunks_tot = e // _CHUNK
    mesh = plsc.VectorSubcoreMesh(
        core_axis_name="c", subcore_axis_name="s",
        num_cores=_NC, num_subcores=_NS)
    sc = pl.kernel(
        functools.partial(_sc_gather_body, n_chunks_tot),
        out_type=(
            jax.ShapeDtypeStruct((e, h), jnp.float32),
            jax.ShapeDtypeStruct((e, h), jnp.float32),
            jax.ShapeDtypeStruct((e,), jnp.float32),
        ),
        mesh=mesh,
        scratch_types=(
            pltpu.VMEM((_CHUNK,), jnp.int32),
            pltpu.VMEM((_CHUNK,), jnp.int32),
            pltpu.VMEM((_CHUNK,), jnp.int32),
            pltpu.VMEM((_CHUNK,), jnp.int32),
            pltpu.VMEM((_CHUNK,), jnp.float32),
            pltpu.VMEM((_CHUNK, h), jnp.float32),
            pltpu.VMEM((_CHUNK, h), jnp.float32),
            pltpu.SemaphoreType.DMA,
        ),
        compiler_params=pltpu.CompilerParams(use_tc_tiling_on_sc=False),
    )
    gu, gv, ng = sc(u, v, pu, pv)

    ii = np.arange(kk)
    pt_np = np.zeros((kk, kk), dtype=np.float32)
    pt_np[(ii % k) * k + ii // k, ii] = 1.0
    pt = jnp.asarray(pt_np)

    be = 2560
    grid = (e // be,)
    out = pl.pallas_call(
        _mlp_body,
        grid=grid,
        in_specs=[
            pl.BlockSpec((be, h), lambda i: (i, 0)),
            pl.BlockSpec((be, h), lambda i: (i, 0)),
            pl.BlockSpec((be, 1), lambda i: (i, 0)),
            pl.BlockSpec((h, h), lambda i: (0, 0)),
            pl.BlockSpec((1, h), lambda i: (0, 0)),
            pl.BlockSpec((h, kk), lambda i: (0, 0)),
            pl.BlockSpec((1, kk), lambda i: (0, 0)),
            pl.BlockSpec((kk, kk), lambda i: (0, 0)),
        ],
        out_specs=pl.BlockSpec((be, kk), lambda i: (i, 0)),
        out_shape=jax.ShapeDtypeStruct((e, kk), jnp.float32),
    )(gu, gv, ng.reshape(e, 1), W2, b2.reshape(1, h), W3,
      b3.reshape(1, kk), pt)

    return out.reshape(e, k, k)
